# baseline (speedup 1.0000x reference)
import functools

import jax
import jax.numpy as jnp
from jax import lax
from jax.experimental import pallas as pl
from jax.experimental.pallas import tpu as pltpu
from jax.experimental.pallas import tpu_sc as plsc

N = 100000
E = 3200000
D = 16

NC = 2
NS = 16
NW = NC * NS
EPAD = 3276800
EPW = EPAD // NW
CHUNK_E = 2560
NCHUNK = EPW // CHUNK_E
G = 128
GPC = CHUNK_E // G

SA = 6256
SA_LO = N - SA

CR = 800
CSTRIPE = 3200
CS_LO = N - CSTRIPE

_mesh = plsc.VectorSubcoreMesh(core_axis_name="c", subcore_axis_name="s")
_params = pltpu.CompilerParams(use_tc_tiling_on_sc=False)


def _bcast_lane(v16, i):
    idx = jnp.full((16, 1), i, dtype=jnp.int32)
    dnums = lax.GatherDimensionNumbers(
        offset_dims=(), collapsed_slice_dims=(0,), start_index_map=(0,))
    return lax.gather(v16, idx, dnums, slice_sizes=(1,),
                      mode=lax.GatherScatterMode.PROMISE_IN_BOUNDS)


@functools.partial(
    pl.kernel,
    out_type=[jax.ShapeDtypeStruct((N, D), jnp.float32),
              jax.ShapeDtypeStruct((N, D), jnp.float32)],
    mesh=_mesh,
    compiler_params=_params,
    scratch_types=[
        pltpu.VMEM((2, CHUNK_E), jnp.int32),
        pltpu.VMEM((2, CHUNK_E), jnp.int32),
        pltpu.VMEM((2, CHUNK_E), jnp.float32),
        pltpu.VMEM((1, GPC, G), jnp.int32),
        pltpu.VMEM((5, G, D), jnp.float32),
        pltpu.VMEM_SHARED((N, D), jnp.float32),
        pltpu.SemaphoreType.DMA,
        pltpu.SemaphoreType.DMA,
        pltpu.SemaphoreType.DMA,
        pltpu.SemaphoreType.DMA,
        pltpu.SemaphoreType.DMA,
        pltpu.SemaphoreType.DMA,
        pltpu.SemaphoreType.DMA,
        pltpu.SemaphoreType.DMA,
        pltpu.SemaphoreType.DMA,
        pltpu.SemaphoreType.DMA,
        pltpu.SemaphoreType.DMA,
    ],
)
def _spmm_sc(ei_hbm, val_hbm, e_hbm, p0_hbm, p1_hbm,
             sbuf, dbuf, vbuf, dst2d, rows, acc,
             tsem, g0, g1, g2, g3, g4, s0, s1, s2, s3, s4):
    cid = lax.axis_index("c")
    sid = lax.axis_index("s")
    wid = cid * NS + sid
    gsem = (g0, g1, g2, g3, g4)
    ssem = (s0, s1, s2, s3, s4)

    stripe = pl.multiple_of(jnp.minimum(sid * SA, SA_LO), 8)
    pltpu.sync_copy(e_hbm.at[pl.ds(stripe, SA)], acc.at[pl.ds(stripe, SA)])
    plsc.subcore_barrier()

    ebase = wid * EPW

    def fire_stage(tt, par):
        off = pl.multiple_of(ebase + tt * CHUNK_E, 8)
        pltpu.async_copy(ei_hbm.at[1, pl.ds(off, CHUNK_E)], sbuf.at[par], tsem)
        pltpu.async_copy(ei_hbm.at[0, pl.ds(off, CHUNK_E)], dbuf.at[par], tsem)
        pltpu.async_copy(val_hbm.at[pl.ds(off, CHUNK_E)], vbuf.at[par], tsem)

    def wait_stage(tt, par):
        off = pl.multiple_of(ebase + tt * CHUNK_E, 8)
        pltpu.make_async_copy(ei_hbm.at[1, pl.ds(off, CHUNK_E)], sbuf.at[par],
                              tsem).wait()
        pltpu.make_async_copy(ei_hbm.at[0, pl.ds(off, CHUNK_E)], dbuf.at[par],
                              tsem).wait()
        pltpu.make_async_copy(val_hbm.at[pl.ds(off, CHUNK_E)], vbuf.at[par],
                              tsem).wait()

    def fire_gather(jj, b, par):
        pltpu.async_copy(e_hbm.at[sbuf.at[par, pl.ds(jj * G, G)]],
                         rows.at[b], gsem[b])

    def wait_gather(jj, b, par):
        pltpu.make_async_copy(e_hbm.at[sbuf.at[par, pl.ds(jj * G, G)]],
                              rows.at[b], gsem[b]).wait()

    def fire_scatter(jj, b, par):
        pltpu.async_copy(rows.at[b], acc.at[dst2d.at[0, jj]], ssem[b],
                         add=True)

    def wait_scatter(jj, b, par):
        pltpu.make_async_copy(rows.at[b], acc.at[dst2d.at[0, jj]],
                              ssem[b]).wait()

    def scale(jj, b, par):
        for q in range(G // 16):
            vals16 = vbuf[par, pl.ds(jj * G + q * 16, 16)]
            for i in range(16):
                r = q * 16 + i
                rows[b, r, :] = rows[b, r, :] * _bcast_lane(vals16, i)

    fire_stage(0, 0)

    @pl.loop(0, NCHUNK, step=2)
    def _chunks(t):
        for par in range(2):
            tt = t + par
            wait_stage(tt, par)

            @pl.when(tt + 1 < NCHUNK)
            def _():
                fire_stage(tt + 1, 1 - par)

            @pl.loop(0, GPC)
            def _mkrows(j):
                for q in range(G // 16):
                    dst2d[0, j, pl.ds(q * 16, 16)] = (
                        dbuf[par, pl.ds(j * G + q * 16, 16)])

            fire_gather(0, 0, par)
            fire_gather(1, 1, par)
            fire_gather(2, 2, par)
            fire_gather(3, 3, par)

            @pl.loop(0, GPC, step=5)
            def _ring(j):
                for b in range(5):
                    jj = j + b
                    wait_gather(jj, b, par)

                    @pl.when(jj >= 1)
                    def _():
                        wait_scatter(jj - 1, (b + 4) % 5, par)

                    @pl.when(jj + 4 < GPC)
                    def _():
                        fire_gather(jj + 4, (b + 4) % 5, par)

                    scale(jj, b, par)
                    fire_scatter(jj, b, par)

            wait_scatter(GPC - 1, (GPC - 1) % 5, par)

    plsc.subcore_barrier()

    @pl.when(cid == 0)
    def _():
        pltpu.sync_copy(acc.at[pl.ds(stripe, SA)], p0_hbm.at[pl.ds(stripe, SA)])

    @pl.when(cid == 1)
    def _():
        pltpu.sync_copy(acc.at[pl.ds(stripe, SA)], p1_hbm.at[pl.ds(stripe, SA)])


def _make_combine(mean):
    @functools.partial(
        pl.kernel,
        out_type=jax.ShapeDtypeStruct((N, D), jnp.float32),
        mesh=_mesh,
        compiler_params=_params,
        scratch_types=[
            pltpu.VMEM((CR, D), jnp.float32),
            pltpu.VMEM((CR, D), jnp.float32),
            pltpu.VMEM((CR, D), jnp.float32),
            pltpu.VMEM((CR, D), jnp.float32),
        ],
    )
    def _combine(a_hbm, b_hbm, c_hbm, o_hbm, av, bv, cv, ov):
        cid = lax.axis_index("c")
        sid = lax.axis_index("s")
        wid = cid * NS + sid
        base = pl.multiple_of(jnp.minimum(wid * CSTRIPE, CS_LO), 8)

        @pl.loop(0, CSTRIPE // CR)
        def _chunk(c):
            off = pl.multiple_of(base + c * CR, 8)
            pltpu.sync_copy(a_hbm.at[pl.ds(off, CR)], av)
            pltpu.sync_copy(b_hbm.at[pl.ds(off, CR)], bv)
            pltpu.sync_copy(c_hbm.at[pl.ds(off, CR)], cv)

            @pl.loop(0, CR)
            def _row(r):
                if mean:
                    ov[r, :] = (av[r, :] + bv[r, :] + cv[r, :]) * (1.0 / 3.0)
                else:
                    ov[r, :] = av[r, :] + bv[r, :] - cv[r, :]

            pltpu.sync_copy(ov, o_hbm.at[pl.ds(off, CR)])

    return _combine


_combine_layer = _make_combine(mean=False)
_combine_mean = _make_combine(mean=True)


def kernel(edge_index, edge_values, embedding_weight):
    pad_idx = jnp.arange(EPAD - E, dtype=jnp.int32) % N
    ei = jnp.concatenate(
        [edge_index.astype(jnp.int32),
         jnp.stack([pad_idx, pad_idx])], axis=1)
    ev = jnp.concatenate(
        [edge_values, jnp.zeros((EPAD - E,), jnp.float32)])
    e0 = embedding_weight

    p0, p1 = _spmm_sc(ei, ev, e0)
    e1 = _combine_layer(p0, p1, e0)
    q0, q1 = _spmm_sc(ei, ev, e1)
    return _combine_mean(e0, q0, q1)


# speedup vs baseline: 1.0756x; 1.0756x over previous
import functools

import jax
import jax.numpy as jnp
from jax import lax
from jax.experimental import pallas as pl
from jax.experimental.pallas import tpu as pltpu
from jax.experimental.pallas import tpu_sc as plsc

N = 100000
E = 3200000
D = 16

NC = 2
NS = 16
NW = NC * NS
EPAD = 3276800
EPW = EPAD // NW
CHUNK_E = 2560
NCHUNK = EPW // CHUNK_E
G = 128
GPC = CHUNK_E // G

SA = 6256
SA_LO = N - SA

CR = 800
CSTRIPE = 3200
CS_LO = N - CSTRIPE

_mesh = plsc.VectorSubcoreMesh(core_axis_name="c", subcore_axis_name="s")
_params = pltpu.CompilerParams(use_tc_tiling_on_sc=False)


def _bcast_lane(v16, i):
    idx = jnp.full((16, 1), i, dtype=jnp.int32)
    dnums = lax.GatherDimensionNumbers(
        offset_dims=(), collapsed_slice_dims=(0,), start_index_map=(0,))
    return lax.gather(v16, idx, dnums, slice_sizes=(1,),
                      mode=lax.GatherScatterMode.PROMISE_IN_BOUNDS)


@functools.partial(
    pl.kernel,
    out_type=[jax.ShapeDtypeStruct((N, D), jnp.float32),
              jax.ShapeDtypeStruct((N, D), jnp.float32)],
    mesh=_mesh,
    compiler_params=_params,
    scratch_types=[
        pltpu.VMEM((2, CHUNK_E), jnp.int32),
        pltpu.VMEM((2, CHUNK_E), jnp.int32),
        pltpu.VMEM((2, CHUNK_E), jnp.float32),
        pltpu.VMEM((1, GPC, G), jnp.int32),
        pltpu.VMEM((5, G, D), jnp.float32),
        pltpu.VMEM_SHARED((N, D), jnp.float32),
        pltpu.SemaphoreType.DMA,
        pltpu.SemaphoreType.DMA,
        pltpu.SemaphoreType.DMA,
        pltpu.SemaphoreType.DMA,
        pltpu.SemaphoreType.DMA,
        pltpu.SemaphoreType.DMA,
        pltpu.SemaphoreType.DMA,
        pltpu.SemaphoreType.DMA,
        pltpu.SemaphoreType.DMA,
        pltpu.SemaphoreType.DMA,
        pltpu.SemaphoreType.DMA,
    ],
)
def _spmm_sc(ei_hbm, val_hbm, e_hbm, p0_hbm, p1_hbm,
             sbuf, dbuf, vbuf, dst2d, rows, acc,
             tsem, g0, g1, g2, g3, g4, s0, s1, s2, s3, s4):
    cid = lax.axis_index("c")
    sid = lax.axis_index("s")
    wid = cid * NS + sid
    gsem = (g0, g1, g2, g3, g4)
    ssem = (s0, s1, s2, s3, s4)

    stripe = pl.multiple_of(jnp.minimum(sid * SA, SA_LO), 8)
    pltpu.sync_copy(e_hbm.at[pl.ds(stripe, SA)], acc.at[pl.ds(stripe, SA)])
    plsc.subcore_barrier()

    ebase = wid * EPW

    def fire_stage(tt, par):
        off = pl.multiple_of(ebase + tt * CHUNK_E, 8)
        pltpu.async_copy(ei_hbm.at[1, pl.ds(off, CHUNK_E)], sbuf.at[par], tsem)
        pltpu.async_copy(ei_hbm.at[0, pl.ds(off, CHUNK_E)], dbuf.at[par], tsem)
        pltpu.async_copy(val_hbm.at[pl.ds(off, CHUNK_E)], vbuf.at[par], tsem)

    def wait_stage(tt, par):
        off = pl.multiple_of(ebase + tt * CHUNK_E, 8)
        pltpu.make_async_copy(ei_hbm.at[1, pl.ds(off, CHUNK_E)], sbuf.at[par],
                              tsem).wait()
        pltpu.make_async_copy(ei_hbm.at[0, pl.ds(off, CHUNK_E)], dbuf.at[par],
                              tsem).wait()
        pltpu.make_async_copy(val_hbm.at[pl.ds(off, CHUNK_E)], vbuf.at[par],
                              tsem).wait()

    def fire_gather(jj, b, par):
        pltpu.async_copy(e_hbm.at[sbuf.at[par, pl.ds(jj * G, G)]],
                         rows.at[b], gsem[b])

    def wait_gather(jj, b, par):
        pltpu.make_async_copy(e_hbm.at[sbuf.at[par, pl.ds(jj * G, G)]],
                              rows.at[b], gsem[b]).wait()

    def fire_scatter(jj, b, par):
        pltpu.async_copy(rows.at[b], acc.at[dst2d.at[0, jj]], ssem[b],
                         add=True)

    def wait_scatter(jj, b, par):
        pltpu.make_async_copy(rows.at[b], acc.at[dst2d.at[0, jj]],
                              ssem[b]).wait()

    def scale(jj, b, par):
        for q in range(G // 16):
            vals16 = vbuf[par, pl.ds(jj * G + q * 16, 16)]
            for i in range(16):
                r = q * 16 + i
                rows[b, r, :] = rows[b, r, :] * _bcast_lane(vals16, i)

    fire_stage(0, 0)

    @pl.loop(0, NCHUNK, step=2)
    def _chunks(t):
        for par in range(2):
            tt = t + par

            @pl.when(ebase + tt * CHUNK_E < E)
            def _():
                wait_stage(tt, par)

                @pl.when((tt + 1 < NCHUNK)
                         & (ebase + (tt + 1) * CHUNK_E < E))
                def _():
                    fire_stage(tt + 1, 1 - par)

                @pl.loop(0, GPC)
                def _mkrows(j):
                    for q in range(G // 16):
                        dst2d[0, j, pl.ds(q * 16, 16)] = (
                            dbuf[par, pl.ds(j * G + q * 16, 16)])

                fire_gather(0, 0, par)
                fire_gather(1, 1, par)
                fire_gather(2, 2, par)

                @pl.loop(0, GPC, step=5)
                def _ring(j):
                    for b in range(5):
                        jj = j + b
                        wait_gather(jj, b, par)

                        @pl.when(jj >= 2)
                        def _():
                            wait_scatter(jj - 2, (b + 3) % 5, par)

                        @pl.when(jj + 3 < GPC)
                        def _():
                            fire_gather(jj + 3, (b + 3) % 5, par)

                        scale(jj, b, par)
                        fire_scatter(jj, b, par)

                wait_scatter(GPC - 2, (GPC - 2) % 5, par)
                wait_scatter(GPC - 1, (GPC - 1) % 5, par)

    plsc.subcore_barrier()

    @pl.when(cid == 0)
    def _():
        pltpu.sync_copy(acc.at[pl.ds(stripe, SA)], p0_hbm.at[pl.ds(stripe, SA)])

    @pl.when(cid == 1)
    def _():
        pltpu.sync_copy(acc.at[pl.ds(stripe, SA)], p1_hbm.at[pl.ds(stripe, SA)])


def _make_combine(mean):
    @functools.partial(
        pl.kernel,
        out_type=jax.ShapeDtypeStruct((N, D), jnp.float32),
        mesh=_mesh,
        compiler_params=_params,
        scratch_types=[
            pltpu.VMEM((CR, D), jnp.float32),
            pltpu.VMEM((CR, D), jnp.float32),
            pltpu.VMEM((CR, D), jnp.float32),
            pltpu.VMEM((CR, D), jnp.float32),
        ],
    )
    def _combine(a_hbm, b_hbm, c_hbm, o_hbm, av, bv, cv, ov):
        cid = lax.axis_index("c")
        sid = lax.axis_index("s")
        wid = cid * NS + sid
        base = pl.multiple_of(jnp.minimum(wid * CSTRIPE, CS_LO), 8)

        @pl.loop(0, CSTRIPE // CR)
        def _chunk(c):
            off = pl.multiple_of(base + c * CR, 8)
            pltpu.sync_copy(a_hbm.at[pl.ds(off, CR)], av)
            pltpu.sync_copy(b_hbm.at[pl.ds(off, CR)], bv)
            pltpu.sync_copy(c_hbm.at[pl.ds(off, CR)], cv)

            @pl.loop(0, CR)
            def _row(r):
                if mean:
                    ov[r, :] = (av[r, :] + bv[r, :] + cv[r, :]) * (1.0 / 3.0)
                else:
                    ov[r, :] = av[r, :] + bv[r, :] - cv[r, :]

            pltpu.sync_copy(ov, o_hbm.at[pl.ds(off, CR)])

    return _combine


_combine_layer = _make_combine(mean=False)
_combine_mean = _make_combine(mean=True)


def kernel(edge_index, edge_values, embedding_weight):
    ei = edge_index.astype(jnp.int32)
    ev = edge_values
    e0 = embedding_weight

    p0, p1 = _spmm_sc(ei, ev, e0)
    e1 = _combine_layer(p0, p1, e0)
    q0, q1 = _spmm_sc(ei, ev, e1)
    return _combine_mean(e0, q0, q1)
